# baseline (speedup 1.0000x reference)
import functools

import jax
import jax.numpy as jnp
from jax import lax
from jax.experimental import pallas as pl
from jax.experimental.pallas import tpu as pltpu
from jax.experimental.pallas import tpu_sc as plsc

_NC = 2
_NS = 16
_NW = _NC * _NS
_L = 16
_B = 8192
_D = 64
_V = 100000
_BLK = 128
_NBF = _V // _BLK
_TAIL = _NBF
_TAILW = _V - _TAIL * _BLK
_RING = 128

_mesh = plsc.VectorSubcoreMesh(core_axis_name="c", subcore_axis_name="s")


def _iota16():
    return lax.iota(jnp.int32, _L)


@functools.partial(
    pl.kernel,
    mesh=_mesh,
    compiler_params=pltpu.CompilerParams(use_tc_tiling_on_sc=True,
                                         needs_layout_passes=False),
    out_type=jax.ShapeDtypeStruct((_B, _D), jnp.float32),
    scratch_types=[
        pltpu.VMEM((_B,), jnp.int32),
        pltpu.VMEM((_B,), jnp.int32),
        pltpu.VMEM((_D, _BLK), jnp.float32),
        pltpu.VMEM((_D, _BLK), jnp.float32),
        pltpu.VMEM((_D, _BLK), jnp.float32),
        pltpu.VMEM((_RING + 1, _D), jnp.float32),
        pltpu.SemaphoreType.DMA,
        pltpu.SemaphoreType.DMA,
        pltpu.SemaphoreType.DMA,
    ],
)
def _sc_gather(idx_hbm, tableT_hbm, tailT_hbm, emb_hbm,
               idx_v, mcode, slab0, slab1, tail_v, rowbuf,
               ssem0, ssem1, wsem):
    wid = lax.axis_index("s") * _NC + lax.axis_index("c")
    lo_b = (wid * _NBF) // _NW
    hi_b = ((wid + 1) * _NBF) // _NW
    owns_tail = wid == (_NW - 1)

    def issue_slab(b, slab, ssem):
        col = pl.multiple_of(b * _BLK, _BLK)
        pltpu.async_copy(tableT_hbm.at[:, pl.ds(col, _BLK)], slab, ssem)

    def wait_slab(slab, ssem):
        pltpu.make_async_copy(tableT_hbm.at[:, pl.ds(0, _BLK)], slab,
                              ssem).wait()

    def wait_row():
        pltpu.make_async_copy(emb_hbm.at[pl.ds(0, 1)],
                              rowbuf.at[pl.ds(0, 1)], wsem).wait()

    issue_slab(lo_b, slab0, ssem0)
    issue_slab(lo_b + 1, slab1, ssem1)

    def credit(_, c):
        pltpu.async_copy(emb_hbm.at[pl.ds(0, 1)],
                         rowbuf.at[pl.ds(_RING, 1)], wsem)
        return c

    lax.fori_loop(0, _RING, credit, jnp.int32(0))

    pltpu.sync_copy(idx_hbm, idx_v)
    mhi = jnp.where(owns_tail, _TAIL + 1, hi_b)

    def mext(v, moff):
        ivec = idx_v[pl.ds(v * _L, _L)]
        bvec = lax.shift_right_logical(ivec, 7)
        m = (bvec >= lo_b) & (bvec < mhi)
        code = ((bvec << 20) | ((ivec & (_BLK - 1)) << 13)
                | (_iota16() + v * _L))
        key = jnp.where(m, 0, 1)
        _, front = plsc.sort_key_val(key, code)
        mcode[pl.ds(moff, _L)] = front
        return moff + jnp.sum(m.astype(jnp.int32))

    nm = lax.fori_loop(0, _B // _L, mext, jnp.int32(0))
    nv = (nm + _L - 1) // _L

    def process(b, slab, hit):
        def scan_body(v, hit):
            cvec = mcode[pl.ds(v * _L, _L)]
            bm = lax.shift_right_logical(cvec, 20)
            inb = (_iota16() + v * _L) < nm
            m = (bm == b) & inb
            k = jnp.sum(m.astype(jnp.int32))
            _, front = plsc.sort_key_val(jnp.where(m, 0, 1), cvec)

            def hitloop(j, hit):
                onehot = _iota16() == j
                code = jnp.sum(jnp.where(onehot, front, 0))
                pos = code & (_B - 1)
                lane = lax.shift_right_logical(code, 13) & (_BLK - 1)
                lanevec = jnp.full((_L,), lane, jnp.int32)
                r = hit & (_RING - 1)
                wait_row()
                for q in range(_D // _L):
                    g = plsc.load_gather(slab,
                                         [_iota16() + q * _L, lanevec])
                    rowbuf[r, pl.ds(q * _L, _L)] = g
                pltpu.async_copy(rowbuf.at[pl.ds(r, 1)],
                                 emb_hbm.at[pl.ds(pos, 1)], wsem)
                return hit + 1

            return lax.fori_loop(0, k, hitloop, hit)

        return lax.fori_loop(0, nv, scan_body, hit)

    npairs = (hi_b - lo_b + 1) // 2
    last_b = hi_b - 1

    def pair_body(p, hit):
        b0 = lo_b + 2 * p
        b1 = b0 + 1
        wait_slab(slab0, ssem0)
        hit = process(b0, slab0, hit)
        issue_slab(jnp.minimum(b0 + 2, last_b), slab0, ssem0)
        wait_slab(slab1, ssem1)
        b1_eff = jnp.where(b1 < hi_b, b1, -1)
        hit = process(b1_eff, slab1, hit)
        issue_slab(jnp.minimum(b1 + 2, last_b), slab1, ssem1)
        return hit

    hit = lax.fori_loop(0, npairs, pair_body, jnp.int32(0))
    wait_slab(slab0, ssem0)
    wait_slab(slab1, ssem1)

    pltpu.sync_copy(tailT_hbm, tail_v)
    tail_b = jnp.where(owns_tail, _TAIL, -1)
    hit = process(tail_b, tail_v, hit)

    def drain(_, c):
        wait_row()
        return c

    lax.fori_loop(0, _RING, drain, hit)


_GB = 1024


def _tc_body(emb_ref, ft_ref, o_ref, acc_ref):
    i = pl.program_id(0)

    @pl.when(i == 0)
    def _():
        acc_ref[0] = 0.0
        acc_ref[1] = 0.0
        acc_ref[2] = 0.0

    emb = emb_ref[...]
    ft = ft_ref[...]
    acc_ref[0] = acc_ref[0] + jnp.sum(emb * emb)
    c = jnp.dot(ft, emb, preferred_element_type=jnp.float32)
    eye = (lax.broadcasted_iota(jnp.int32, (_D, _D), 0)
           == lax.broadcasted_iota(jnp.int32, (_D, _D), 1))
    acc_ref[1] = acc_ref[1] + jnp.sum(jnp.where(eye, c, 0.0))
    acc_ref[2] = acc_ref[2] + jnp.sum(ft * ft)

    @pl.when(i == pl.num_programs(0) - 1)
    def _():
        e2 = acc_ref[0]
        dt = acc_ref[1]
        f2 = acc_ref[2]
        o_ref[0, 0] = 2.0 * f2 - 2.0 * jnp.sqrt(f2 / e2) * dt


_combine = pl.pallas_call(
    _tc_body,
    grid=(_B // _GB,),
    in_specs=[pl.BlockSpec((_GB, _D), lambda i: (i, 0)),
              pl.BlockSpec((_D, _GB), lambda i: (0, i))],
    out_specs=pl.BlockSpec((1, 1), lambda i: (0, 0),
                           memory_space=pltpu.SMEM),
    out_shape=jax.ShapeDtypeStruct((1, 1), jnp.float32),
    scratch_shapes=[pltpu.SMEM((3,), jnp.float32)],
)


def kernel(feature, item, neigh, pca_emb):
    idx = jnp.concatenate([item, neigh]).astype(jnp.int32)
    tail = jnp.pad(pca_emb[_TAIL * _BLK:], ((0, _BLK - _TAILW), (0, 0)))
    emb = _sc_gather(idx, pca_emb.T, tail.T)
    out = _combine(emb, feature.T)
    return out[0, 0]


# speedup vs baseline: 3.5635x; 3.5635x over previous
import functools

import jax
import jax.numpy as jnp
from jax import lax
from jax.experimental import pallas as pl
from jax.experimental.pallas import tpu as pltpu
from jax.experimental.pallas import tpu_sc as plsc

_NC = 2
_NS = 16
_NW = _NC * _NS
_L = 16
_B = 8192
_D = 64
_BPW = _B // _NW

_mesh = plsc.VectorSubcoreMesh(core_axis_name="c", subcore_axis_name="s")


@functools.partial(
    pl.kernel,
    mesh=_mesh,
    compiler_params=pltpu.CompilerParams(use_tc_tiling_on_sc=True),
    out_type=(
        jax.ShapeDtypeStruct((_NW, _L), jnp.float32),
        jax.ShapeDtypeStruct((_NW, _L), jnp.float32),
        jax.ShapeDtypeStruct((_NW, _L), jnp.float32),
    ),
    scratch_types=[
        pltpu.VMEM((_BPW,), jnp.int32),
        pltpu.VMEM((_BPW, _D), jnp.float32),
        pltpu.VMEM((_BPW, _D), jnp.float32),
        pltpu.VMEM((3, _L), jnp.float32),
        pltpu.SemaphoreType.DMA,
        pltpu.SemaphoreType.DMA,
    ],
)
def _sc_partials(idx_hbm, feat_hbm, table_hbm, e2_hbm, dt_hbm, f2_hbm,
                 idx_v, rows_v, feat_v, acc_v, gsem, fsem):
    wid = lax.axis_index("s") * _NC + lax.axis_index("c")
    base = wid * _BPW
    pltpu.sync_copy(idx_hbm.at[pl.ds(base, _BPW)], idx_v)
    fcopy = pltpu.async_copy(feat_hbm.at[pl.ds(base, _BPW)], feat_v, fsem)

    def fire(k, carry):
        iv = idx_v[pl.ds(k * _L, _L)]
        for j in range(_L):
            di = iv[j]
            pltpu.async_copy(table_hbm.at[pl.ds(di, 1)],
                             rows_v.at[pl.ds(k * _L + j, 1)], gsem)
        return carry

    lax.fori_loop(0, _BPW // _L, fire, 0)
    pltpu.make_async_copy(table_hbm.at[pl.ds(0, _BPW)], rows_v, gsem).wait()
    fcopy.wait()

    zeros = jnp.zeros((_L,), jnp.float32)

    def body(i, carry):
        e2, dt, f2 = carry
        for j in range(_D // _L):
            r = rows_v[i, pl.ds(j * _L, _L)]
            f = feat_v[i, pl.ds(j * _L, _L)]
            e2 = e2 + r * r
            dt = dt + r * f
            f2 = f2 + f * f
        return (e2, dt, f2)

    e2, dt, f2 = lax.fori_loop(0, _BPW, body, (zeros, zeros, zeros))
    acc_v[0, :] = e2
    acc_v[1, :] = dt
    acc_v[2, :] = f2
    pltpu.sync_copy(acc_v.at[0], e2_hbm.at[wid])
    pltpu.sync_copy(acc_v.at[1], dt_hbm.at[wid])
    pltpu.sync_copy(acc_v.at[2], f2_hbm.at[wid])


def _combine_body(e2_ref, dt_ref, f2_ref, o_ref):
    e2 = jnp.sum(e2_ref[...])
    dt = jnp.sum(dt_ref[...])
    f2 = jnp.sum(f2_ref[...])
    o_ref[0, 0] = 2.0 * f2 - 2.0 * jnp.sqrt(f2 / e2) * dt


_combine = pl.pallas_call(
    _combine_body,
    out_shape=jax.ShapeDtypeStruct((1, 1), jnp.float32),
    out_specs=pl.BlockSpec(memory_space=pltpu.SMEM),
)


def kernel(feature, item, neigh, pca_emb):
    idx = jnp.concatenate([item, neigh]).astype(jnp.int32)
    e2p, dtp, f2p = _sc_partials(idx, feature, pca_emb)
    out = _combine(e2p, dtp, f2p)
    return out[0, 0]
